# Initial kernel scaffold; baseline (speedup 1.0000x reference)
import jax
import jax.numpy as jnp
from jax.experimental import pallas as pl


def kernel(pos, atom_types, bonded_edge_index, radial_edge_index, c_noise, atom_emb, bond_emb, w_noise0, Wr0, wsh0, Wproj0, Wself0, Wr, wsh, Wout, Wself, noise_w, skip_w, W_head, gain):
    raise NotImplementedError("write your pallas kernel here")


# speedup vs baseline: 5.8796x; 5.8796x over previous
import jax
import jax.numpy as jnp
from jax import lax
from jax.experimental import pallas as pl
from jax.experimental.pallas import tpu as pltpu
from jax.experimental.pallas import tpu_sc as plsc

_N = 50000
_EB = 100000
_ER = 800000
_E = 900000
_NPAD = 51200
_EPAD = 901120
_CUT = 5.0
_NC, _NS = 2, 16

_EPW = _EPAD // (_NC * _NS)
_NPW = _NPAD // (_NC * _NS)
_EPT_FULL = _EPAD // _NS
_EPT_HALF = _EPAD // (2 * _NS)


def _mesh():
    return plsc.VectorSubcoreMesh(
        core_axis_name="c", subcore_axis_name="s",
        num_cores=_NC, num_subcores=_NS)


def _sc_prep(pos16, emb2, srcp, dstp, atp):
    def body(pos_h, emb_h, src_h, dst_h, atp_h, exyzT_h, t0_h,
             sidx, didx, ps, pd, st16, aidx, rowsA, rowsB,
             sem, pl0, pl1, pg0, pg1):
        c = lax.axis_index("c")
        s = lax.axis_index("s")
        w = s * _NC + c
        slp = (pl0, pl1)
        sgp = (pg0, pg1)
        nch = _EPW // 128

        def e_of(i):
            return w * _EPW + i * 128

        def start_load(sp, e0):
            pltpu.async_copy(src_h.at[pl.ds(e0, 128)], sidx.at[sp], slp[sp])
            pltpu.async_copy(dst_h.at[pl.ds(e0, 128)], didx.at[sp], slp[sp])

        def wait_load(sp, e0):
            pltpu.make_async_copy(src_h.at[pl.ds(e0, 128)], sidx.at[sp],
                                  slp[sp]).wait()
            pltpu.make_async_copy(dst_h.at[pl.ds(e0, 128)], didx.at[sp],
                                  slp[sp]).wait()

        def start_gather(sp):
            pltpu.async_copy(pos_h.at[sidx.at[sp]], ps.at[sp], sgp[sp])
            pltpu.async_copy(pos_h.at[didx.at[sp]], pd.at[sp], sgp[sp])

        def wait_gather(sp):
            pltpu.make_async_copy(pos_h.at[sidx.at[sp]], ps.at[sp],
                                  sgp[sp]).wait()
            pltpu.make_async_copy(pos_h.at[didx.at[sp]], pd.at[sp],
                                  sgp[sp]).wait()

        def compute_out(sp, e0):
            def sub8(r8, car2):
                r0 = r8 * 8
                for j in range(8):
                    st16[r0 + j, :] = ps[sp, r0 + j, :] - pd[sp, r0 + j, :]
                return car2

            lax.fori_loop(0, 16, sub8, 0)
            pltpu.sync_copy(st16, exyzT_h.at[pl.ds(e0, 128)])

        start_load(0, e_of(0))
        wait_load(0, e_of(0))
        start_gather(0)
        start_load(1, e_of(1))

        def step(i, car):
            def iter_for(sp, ot):
                wait_gather(sp)

                @pl.when(i < nch - 1)
                def _():
                    wait_load(ot, e_of(i + 1))
                    start_gather(ot)

                compute_out(sp, e_of(i))

                @pl.when(i < nch - 2)
                def _():
                    start_load(sp, e_of(i + 2))

            @pl.when(i % 2 == 0)
            def _():
                iter_for(0, 1)

            @pl.when(i % 2 == 1)
            def _():
                iter_for(1, 0)

            return car

        lax.fori_loop(0, nch, step, 0)

        def nchunk(k, car):
            n0 = w * _NPW + k * 160
            pltpu.sync_copy(atp_h.at[pl.ds(n0, 160)], aidx)
            pltpu.async_copy(emb_h.at[aidx], rowsA, sem).wait()
            for j in range(10):
                aidx[pl.ds(j * 16, 16)] = aidx[pl.ds(j * 16, 16)] + 128
            pltpu.async_copy(emb_h.at[aidx], rowsB, sem).wait()
            pltpu.sync_copy(rowsA, t0_h.at[pl.ds(n0, 160)])
            pltpu.sync_copy(rowsB, t0_h.at[pl.ds(_NPAD + n0, 160)])
            return car

        lax.fori_loop(0, _NPW // 160, nchunk, 0)

    f = pl.kernel(
        body,
        out_type=(jax.ShapeDtypeStruct((_EPAD, 16), jnp.float32),
                  jax.ShapeDtypeStruct((2 * _NPAD, 16), jnp.float32)),
        mesh=_mesh(),
        compiler_params=pltpu.CompilerParams(use_tc_tiling_on_sc=False),
        scratch_types=[
            pltpu.VMEM((2, 128), jnp.int32),
            pltpu.VMEM((2, 128), jnp.int32),
            pltpu.VMEM((2, 128, 16), jnp.float32),
            pltpu.VMEM((2, 128, 16), jnp.float32),
            pltpu.VMEM((128, 16), jnp.float32),
            pltpu.VMEM((160,), jnp.int32),
            pltpu.VMEM((160, 16), jnp.float32),
            pltpu.VMEM((160, 16), jnp.float32),
            pltpu.SemaphoreType.DMA,
            pltpu.SemaphoreType.DMA,
            pltpu.SemaphoreType.DMA,
            pltpu.SemaphoreType.DMA,
            pltpu.SemaphoreType.DMA,
        ],
    )
    return f(pos16, emb2, srcp, dstp, atp)


def _sc_conv(tfull, wt, srcp, dstp, thalf=None):
    has_half = thalf is not None

    def body(*refs):
        if has_half:
            (tf, wf, src_h, dst_h, th, aggF, aggH,
             sidx, didx, rows, wv, mv, acc,
             sem_l0, sem_l1, sem_g0, sem_g1) = refs
        else:
            (tf, wf, src_h, dst_h, aggF,
             sidx, didx, rows, wv, mv, acc,
             sem_l0, sem_l1, sem_g0, sem_g1) = refs
        c = lax.axis_index("c")
        s = lax.axis_index("s")
        z16 = jnp.zeros((16,), jnp.float32)
        for r in range(128):
            mv[r, :] = z16

        def zc(k, car):
            pltpu.sync_copy(mv.at[pl.ds(0, 128)],
                            acc.at[pl.ds(s * 6400 + k * 128, 128)])
            return car

        lax.fori_loop(0, 50, zc, 0)
        plsc.subcore_barrier()

        sl = (sem_l0, sem_l1)
        sg = (sem_g0, sem_g1)

        def run_pass(nsup, e_base, tbl, col_off, t_off, acc_off):
            def e_of(i):
                return e_base + i * 256

            def start_load(sp, e0):
                pltpu.async_copy(src_h.at[pl.ds(e0, 256)], sidx.at[sp],
                                 sl[sp])
                for j in range(2):
                    pltpu.async_copy(dst_h.at[pl.ds(e0 + j * 128, 128)],
                                     didx.at[sp, j], sl[sp])
                pltpu.async_copy(wf.at[pl.ds(e0, 256), pl.ds(col_off, 16)],
                                 wv.at[sp], sl[sp])

            def wait_load(sp, e0):
                pltpu.make_async_copy(src_h.at[pl.ds(e0, 256)], sidx.at[sp],
                                      sl[sp]).wait()
                for j in range(2):
                    pltpu.make_async_copy(dst_h.at[pl.ds(e0 + j * 128, 128)],
                                          didx.at[sp, j], sl[sp]).wait()
                pltpu.make_async_copy(
                    wf.at[pl.ds(e0, 256), pl.ds(col_off, 16)],
                    wv.at[sp], sl[sp]).wait()

            def adjust(sp):
                if t_off is not None:
                    for j in range(16):
                        sidx[sp, pl.ds(j * 16, 16)] = (
                            sidx[sp, pl.ds(j * 16, 16)] + t_off)
                if acc_off:
                    for j in range(2):
                        for k in range(8):
                            didx[sp, j, pl.ds(k * 16, 16)] = (
                                didx[sp, j, pl.ds(k * 16, 16)] + acc_off)

            def start_gather(sp):
                for j in range(2):
                    pltpu.async_copy(
                        tbl.at[sidx.at[sp, pl.ds(j * 128, 128)]],
                        rows.at[sp, pl.ds(j * 128, 128)], sg[sp])

            def wait_gather(sp):
                for j in range(2):
                    pltpu.make_async_copy(
                        tbl.at[sidx.at[sp, pl.ds(j * 128, 128)]],
                        rows.at[sp, pl.ds(j * 128, 128)], sg[sp]).wait()

            def compute_scatter(sp):
                def mul8(r8, car2):
                    r0 = r8 * 8
                    for j in range(8):
                        mv[r0 + j, :] = (rows[sp, r0 + j, :]
                                         * wv[sp, r0 + j, :])
                    return car2

                lax.fori_loop(0, 32, mul8, 0)
                for j in range(2):
                    pltpu.sync_copy(mv.at[pl.ds(j * 128, 128)],
                                    acc.at[didx.at[sp, j]], add=True)

            start_load(0, e_of(0))
            wait_load(0, e_of(0))
            adjust(0)
            start_gather(0)
            if nsup > 1:
                start_load(1, e_of(1))

            def step(i, car):
                def iter_for(sp, ot):
                    wait_gather(sp)

                    @pl.when(i < nsup - 1)
                    def _():
                        wait_load(ot, e_of(i + 1))
                        adjust(ot)
                        start_gather(ot)

                    compute_scatter(sp)

                    @pl.when(i < nsup - 2)
                    def _():
                        start_load(sp, e_of(i + 2))

                @pl.when(i % 2 == 0)
                def _():
                    iter_for(0, 1)

                @pl.when(i % 2 == 1)
                def _():
                    iter_for(1, 0)

                return car

            lax.fori_loop(0, nsup, step, 0)

        fcol = c * 32 if has_half else c * 16
        run_pass(_EPT_FULL // 256, s * _EPT_FULL, tf, fcol, c * _NPAD, 0)
        if has_half:
            run_pass(_EPT_HALF // 256, c * (_EPAD // 2) + s * _EPT_HALF,
                     th, 16, None, _NPAD)
        plsc.subcore_barrier()
        pltpu.sync_copy(acc.at[pl.ds(s * 3200, 3200)],
                        aggF.at[c, pl.ds(s * 3200, 3200)])
        if has_half:
            pltpu.sync_copy(acc.at[pl.ds(_NPAD + s * 3200, 3200)],
                            aggH.at[c, pl.ds(s * 3200, 3200)])

    outs = [jax.ShapeDtypeStruct((2, _NPAD, 16), jnp.float32)]
    if has_half:
        outs.append(jax.ShapeDtypeStruct((2, _NPAD, 16), jnp.float32))
    f = pl.kernel(
        body,
        out_type=tuple(outs) if has_half else outs[0],
        mesh=_mesh(),
        compiler_params=pltpu.CompilerParams(use_tc_tiling_on_sc=False),
        scratch_types=[
            pltpu.VMEM((2, 256), jnp.int32),
            pltpu.VMEM((2, 2, 128), jnp.int32),
            pltpu.VMEM((2, 256, 16), jnp.float32),
            pltpu.VMEM((2, 256, 16), jnp.float32),
            pltpu.VMEM((256, 16), jnp.float32),
            pltpu.VMEM_SHARED((2 * _NPAD, 16), jnp.float32),
            pltpu.SemaphoreType.DMA,
            pltpu.SemaphoreType.DMA,
            pltpu.SemaphoreType.DMA,
            pltpu.SemaphoreType.DMA,
        ],
    )
    if has_half:
        return f(tfull, wt, srcp, dstp, thalf)
    return f(tfull, wt, srcp, dstp)


_TN = _EPAD // 128
_BT = 32


def _tc_feats(exyz3, be8):
    def body(e_ref, be_ref, f_ref):
        i = pl.program_id(0)
        x = e_ref[0]
        y = e_ref[1]
        z = e_ref[2]
        r2 = x * x + y * y + z * z + 1e-18
        rinv = lax.rsqrt(r2)
        r = r2 * rinv
        ux = x * rinv
        uy = y * rinv
        uz = z * rinv
        eg = (i * (_BT * 128)
              + lax.broadcasted_iota(jnp.int32, (_BT, 128), 0) * 128
              + lax.broadcasted_iota(jnp.int32, (_BT, 128), 1))
        isb = (eg >= _ER).astype(jnp.float32)
        vm = (eg < _E).astype(jnp.float32)
        be = be_ref[...]
        for k in range(8):
            f_ref[:, k, :] = be[0, k] + isb * (be[1, k] - be[0, k])
        cutm = (r < _CUT).astype(jnp.float32) * (1.0 / 1.12)
        step = _CUT / 9.0
        for k in range(8):
            vk = _CUT * (k + 1) / 9.0
            dd = (r - vk) * (1.0 / step)
            f_ref[:, 8 + k, :] = jnp.exp(-dd * dd) * cutm
        s3 = 3.0 ** 0.5
        s5 = 5.0 ** 0.5
        s15 = 15.0 ** 0.5
        shs = [vm, s3 * ux * vm, s3 * uy * vm, s3 * uz * vm,
               s15 * ux * uy * vm, s15 * uy * uz * vm,
               (s5 / 2.0) * (3.0 * uz * uz - 1.0) * vm,
               s15 * ux * uz * vm,
               (s15 / 2.0) * (ux * ux - uy * uy) * vm]
        for m, p in enumerate(shs):
            f_ref[:, 16 + m, :] = p
        zz = jnp.zeros((_BT, 128), jnp.float32)
        for f in range(25, 32):
            f_ref[:, f, :] = zz

    return pl.pallas_call(
        body,
        grid=(_TN // _BT,),
        in_specs=[pl.BlockSpec((3, _BT, 128), lambda i: (0, i, 0)),
                  pl.BlockSpec((8, 128), lambda i: (0, 0))],
        out_specs=pl.BlockSpec((_BT, 32, 128), lambda i: (i, 0, 0)),
        out_shape=jax.ShapeDtypeStruct((_TN, 32, 128), jnp.float32),
    )(exyz3, be8)


_BE = 2048


def _tc_w(featsR, Ws, Vs):
    def body(f_ref, w0w, w1w, w2w, w3w, v0w, v1w, v2w, v3w,
             o0, o1, o2, o3):
        blk = f_ref[...]
        for ww, vv, oo in ((w0w, v0w, o0), (w1w, v1w, o1),
                           (w2w, v2w, o2), (w3w, v3w, o3)):
            rw = jnp.dot(blk, ww[...], preferred_element_type=jnp.float32)
            gg = jnp.dot(blk, vv[...], preferred_element_type=jnp.float32)
            oo[...] = rw * gg

    def wspec(d):
        return pl.BlockSpec((32, d), lambda i: (0, 0))

    def ospec(d):
        return pl.BlockSpec((_BE, d), lambda i: (i, 0))

    return pl.pallas_call(
        body,
        grid=(_EPAD // _BE,),
        in_specs=[pl.BlockSpec((_BE, 32), lambda i: (i, 0)),
                  wspec(32), wspec(48), wspec(48), wspec(48),
                  wspec(32), wspec(48), wspec(48), wspec(48)],
        out_specs=[ospec(32), ospec(48), ospec(48), ospec(48)],
        out_shape=[jax.ShapeDtypeStruct((_EPAD, 32), jnp.float32),
                   jax.ShapeDtypeStruct((_EPAD, 48), jnp.float32),
                   jax.ShapeDtypeStruct((_EPAD, 48), jnp.float32),
                   jax.ShapeDtypeStruct((_EPAD, 48), jnp.float32)],
    )(featsR, Ws[0], Ws[1], Ws[2], Ws[3], Vs[0], Vs[1], Vs[2], Vs[3])


_BN = 512


def _tc_upd0(aggF0, t0r, Wp, Ws0s):
    def body(aF_ref, t0_ref, wp_ref, ws_ref, hf_ref, th_ref):
        aF = aF_ref[...]
        t0 = t0_ref[...]
        a32 = jnp.concatenate([aF[0], aF[1]], axis=1)
        h032 = jnp.concatenate([t0[0], t0[1]], axis=1)
        pre = (jnp.dot(a32, wp_ref[...], preferred_element_type=jnp.float32)
               + jnp.dot(h032, ws_ref[...], preferred_element_type=jnp.float32))
        h = pre * (1.0 / (1.0 + jnp.exp(-pre)))
        hf_ref[0] = h[:, 0:16]
        hf_ref[1] = jnp.concatenate(
            [h[:, 32:44], jnp.zeros((_BN, 4), jnp.float32)], axis=1)
        th_ref[...] = h[:, 16:32]

    big = pl.BlockSpec((2, _BN, 16), lambda i: (0, i, 0))
    sml = pl.BlockSpec((_BN, 16), lambda i: (i, 0))
    return pl.pallas_call(
        body,
        grid=(_NPAD // _BN,),
        in_specs=[big, big,
                  pl.BlockSpec((32, 44), lambda i: (0, 0)),
                  pl.BlockSpec((32, 44), lambda i: (0, 0))],
        out_specs=[big, sml],
        out_shape=[jax.ShapeDtypeStruct((2, _NPAD, 16), jnp.float32),
                   jax.ShapeDtypeStruct((_NPAD, 16), jnp.float32)],
    )(aggF0, t0r, Wp, Ws0s)


def _tc_updl(aF, aH, hf, th, Wo, Wss, skl, final, Whg=None):
    def body(aF_ref, aH_ref, hf_ref, th_ref, wo_ref, ws_ref, sk_ref, *outs):
        aFv = aF_ref[...]
        aHv = aH_ref[...]
        hfv = hf_ref[...]
        thv = th_ref[...]
        h = jnp.concatenate([hfv[0], thv, hfv[1][:, 0:12]], axis=1)
        a = jnp.concatenate([aFv[0], aHv[0] + aHv[1], aFv[1][:, 0:12]], axis=1)
        pre = (jnp.dot(a, wo_ref[...], preferred_element_type=jnp.float32)
               + jnp.dot(h, ws_ref[...], preferred_element_type=jnp.float32))
        new = pre * (1.0 / (1.0 + jnp.exp(-pre)))
        hn = h + sk_ref[...][0:1, :] * new
        if final:
            wh_ref, out_ref = outs[0], outs[1]
            out_ref[...] = jnp.dot(hn, wh_ref[...],
                                   preferred_element_type=jnp.float32)
        else:
            hfo_ref, tho_ref = outs[0], outs[1]
            hfo_ref[0] = hn[:, 0:16]
            hfo_ref[1] = jnp.concatenate(
                [hn[:, 32:44], jnp.zeros((_BN, 4), jnp.float32)], axis=1)
            tho_ref[...] = hn[:, 16:32]

    big = pl.BlockSpec((2, _BN, 16), lambda i: (0, i, 0))
    sml = pl.BlockSpec((_BN, 16), lambda i: (i, 0))
    full44 = pl.BlockSpec((44, 44), lambda i: (0, 0))
    in_specs = [big, big, big, sml, full44, full44,
                pl.BlockSpec((8, 44), lambda i: (0, 0))]
    if final:
        in_specs.append(pl.BlockSpec((44, 8), lambda i: (0, 0)))
        return pl.pallas_call(
            body,
            grid=(_NPAD // _BN,),
            in_specs=in_specs,
            out_specs=[pl.BlockSpec((_BN, 8), lambda i: (i, 0))],
            out_shape=[jax.ShapeDtypeStruct((_NPAD, 8), jnp.float32)],
        )(aF, aH, hf, th, Wo, Wss, skl, Whg)[0]
    return pl.pallas_call(
        body,
        grid=(_NPAD // _BN,),
        in_specs=in_specs,
        out_specs=[big, sml],
        out_shape=[jax.ShapeDtypeStruct((2, _NPAD, 16), jnp.float32),
                   jax.ShapeDtypeStruct((_NPAD, 16), jnp.float32)],
    )(aF, aH, hf, th, Wo, Wss, skl)


def kernel(pos, atom_types, bonded_edge_index, radial_edge_index, c_noise,
           atom_emb, bond_emb, w_noise0, Wr0, wsh0, Wproj0, Wself0, Wr, wsh,
           Wout, Wself, noise_w, skip_w, W_head, gain):
    f32 = jnp.float32
    cn = c_noise[0]
    src = jnp.concatenate([radial_edge_index[0],
                           bonded_edge_index[0]]).astype(jnp.int32)
    dst = jnp.concatenate([radial_edge_index[1],
                           bonded_edge_index[1]]).astype(jnp.int32)
    srcp = jnp.pad(src, (0, _EPAD - _E))
    dstp = jnp.pad(dst, (0, _EPAD - _E))
    atp = jnp.pad(atom_types.astype(jnp.int32), (0, _NPAD - _N))
    pos16 = jnp.pad(pos.astype(f32), ((0, 0), (0, 13)))
    embA = jnp.pad(atom_emb[:, 0:16], ((0, 128 - 119), (0, 0)))
    embB = jnp.pad(atom_emb[:, 16:32], ((0, 128 - 119), (0, 0)))
    emb2 = jnp.concatenate([embA, embB], axis=0)

    s0 = 1.0 + cn * w_noise0
    sl = 1.0 + cn * noise_w
    WrS0 = jnp.pad(Wr0 * s0[None, :], ((0, 16), (0, 0)))
    WrSl = jnp.pad(Wr * sl[:, None, :],
                   ((0, 0), (0, 16), (0, 4)))
    Ws = [WrS0, WrSl[0], WrSl[1], WrSl[2]]
    vsh_all = [wsh0[:, 0], wsh[0][:, 0], wsh[1][:, 0], wsh[2][:, 0]]
    Vs = []
    for li, d in enumerate((32, 48, 48, 48)):
        v32 = jnp.pad(vsh_all[li], (16, 7))
        Vs.append(jnp.broadcast_to(v32[:, None], (32, d)))
    be8 = jnp.zeros((8, 128), f32).at[0:2, 0:8].set(bond_emb)
    Ws0s = s0[:, None] * Wself0
    WselfS = sl[:, :, None] * Wself
    sks = jax.nn.sigmoid(cn * skip_w)
    Whg = jnp.pad(W_head * gain, ((0, 0), (0, 5)))

    exyzT, t0full = _sc_prep(pos16, emb2, srcp, dstp, atp)
    exyz3 = exyzT[:, 0:3].T.reshape(3, _TN, 128)
    feats = _tc_feats(exyz3, be8)
    featsR = feats.transpose(0, 2, 1).reshape(_EPAD, 32)
    w0t, w1t, w2t, w3t = _tc_w(featsR, Ws, Vs)
    aggF0 = _sc_conv(t0full, w0t, srcp, dstp)
    hf, th = _tc_upd0(aggF0, t0full.reshape(2, _NPAD, 16), Wproj0, Ws0s)
    out8 = None
    for l, wlt in enumerate((w1t, w2t, w3t)):
        aF, aH = _sc_conv(hf.reshape(2 * _NPAD, 16), wlt, srcp, dstp, th)
        skl = jnp.zeros((8, 44), f32).at[0].set(sks[l])
        if l < 2:
            hf, th = _tc_updl(aF, aH, hf, th, Wout[l], WselfS[l], skl,
                              final=False)
        else:
            out8 = _tc_updl(aF, aH, hf, th, Wout[l], WselfS[l], skl,
                            final=True, Whg=Whg)
    return out8[:_N, 0:3]


# speedup vs baseline: 6.1577x; 1.0473x over previous
import jax
import jax.numpy as jnp
from jax import lax
from jax.experimental import pallas as pl
from jax.experimental.pallas import tpu as pltpu
from jax.experimental.pallas import tpu_sc as plsc

_N = 50000
_EB = 100000
_ER = 800000
_E = 900000
_NPAD = 51200
_EPAD = 901120
_CUT = 5.0
_NC, _NS = 2, 16

_EPW = _EPAD // (_NC * _NS)
_NPW = _NPAD // (_NC * _NS)
_EPT_FULL = _EPAD // _NS
_EPT_HALF = _EPAD // (2 * _NS)


def _mesh():
    return plsc.VectorSubcoreMesh(
        core_axis_name="c", subcore_axis_name="s",
        num_cores=_NC, num_subcores=_NS)


def _sc_prep(pos16, emb2, srcp, dstp, atp):
    def body(pos_h, emb_h, src_h, dst_h, atp_h, exyzT_h, t0_h,
             sidx, didx, ps, pd, st16, aidx, rowsA, rowsB,
             sem, pl0, pl1, pg0, pg1):
        c = lax.axis_index("c")
        s = lax.axis_index("s")
        w = s * _NC + c
        slp = (pl0, pl1)
        sgp = (pg0, pg1)
        nch = _EPW // 128

        def e_of(i):
            return w * _EPW + i * 128

        def start_load(sp, e0):
            pltpu.async_copy(src_h.at[pl.ds(e0, 128)], sidx.at[sp], slp[sp])
            pltpu.async_copy(dst_h.at[pl.ds(e0, 128)], didx.at[sp], slp[sp])

        def wait_load(sp, e0):
            pltpu.make_async_copy(src_h.at[pl.ds(e0, 128)], sidx.at[sp],
                                  slp[sp]).wait()
            pltpu.make_async_copy(dst_h.at[pl.ds(e0, 128)], didx.at[sp],
                                  slp[sp]).wait()

        def start_gather(sp):
            pltpu.async_copy(pos_h.at[sidx.at[sp]], ps.at[sp], sgp[sp])
            pltpu.async_copy(pos_h.at[didx.at[sp]], pd.at[sp], sgp[sp])

        def wait_gather(sp):
            pltpu.make_async_copy(pos_h.at[sidx.at[sp]], ps.at[sp],
                                  sgp[sp]).wait()
            pltpu.make_async_copy(pos_h.at[didx.at[sp]], pd.at[sp],
                                  sgp[sp]).wait()

        def compute_out(sp, e0):
            def sub8(r8, car2):
                r0 = r8 * 8
                for j in range(8):
                    st16[r0 + j, :] = ps[sp, r0 + j, :] - pd[sp, r0 + j, :]
                return car2

            lax.fori_loop(0, 16, sub8, 0)
            pltpu.sync_copy(st16, exyzT_h.at[pl.ds(e0, 128)])

        start_load(0, e_of(0))
        wait_load(0, e_of(0))
        start_gather(0)
        start_load(1, e_of(1))

        def step(i, car):
            def iter_for(sp, ot):
                wait_gather(sp)

                @pl.when(i < nch - 1)
                def _():
                    wait_load(ot, e_of(i + 1))
                    start_gather(ot)

                compute_out(sp, e_of(i))

                @pl.when(i < nch - 2)
                def _():
                    start_load(sp, e_of(i + 2))

            @pl.when(i % 2 == 0)
            def _():
                iter_for(0, 1)

            @pl.when(i % 2 == 1)
            def _():
                iter_for(1, 0)

            return car

        lax.fori_loop(0, nch, step, 0)

        def nchunk(k, car):
            n0 = w * _NPW + k * 160
            pltpu.sync_copy(atp_h.at[pl.ds(n0, 160)], aidx)
            pltpu.async_copy(emb_h.at[aidx], rowsA, sem).wait()
            for j in range(10):
                aidx[pl.ds(j * 16, 16)] = aidx[pl.ds(j * 16, 16)] + 128
            pltpu.async_copy(emb_h.at[aidx], rowsB, sem).wait()
            pltpu.sync_copy(rowsA, t0_h.at[pl.ds(n0, 160)])
            pltpu.sync_copy(rowsB, t0_h.at[pl.ds(_NPAD + n0, 160)])
            return car

        lax.fori_loop(0, _NPW // 160, nchunk, 0)

    f = pl.kernel(
        body,
        out_type=(jax.ShapeDtypeStruct((_EPAD, 16), jnp.float32),
                  jax.ShapeDtypeStruct((2 * _NPAD, 16), jnp.float32)),
        mesh=_mesh(),
        compiler_params=pltpu.CompilerParams(use_tc_tiling_on_sc=False),
        scratch_types=[
            pltpu.VMEM((2, 128), jnp.int32),
            pltpu.VMEM((2, 128), jnp.int32),
            pltpu.VMEM((2, 128, 16), jnp.float32),
            pltpu.VMEM((2, 128, 16), jnp.float32),
            pltpu.VMEM((128, 16), jnp.float32),
            pltpu.VMEM((160,), jnp.int32),
            pltpu.VMEM((160, 16), jnp.float32),
            pltpu.VMEM((160, 16), jnp.float32),
            pltpu.SemaphoreType.DMA,
            pltpu.SemaphoreType.DMA,
            pltpu.SemaphoreType.DMA,
            pltpu.SemaphoreType.DMA,
            pltpu.SemaphoreType.DMA,
        ],
    )
    return f(pos16, emb2, srcp, dstp, atp)


def _sc_conv(tfull, wt, srcp, dstp, thalf=None):
    has_half = thalf is not None

    def body(*refs):
        if has_half:
            (tf, wf, src_h, dst_h, th, aggF, aggH,
             sidx, didx, didx_s, rows, wv, mv, acc,
             sem_l0, sem_l1, sem_g0, sem_g1, sem_s0, sem_s1) = refs
        else:
            (tf, wf, src_h, dst_h, aggF,
             sidx, didx, didx_s, rows, wv, mv, acc,
             sem_l0, sem_l1, sem_g0, sem_g1, sem_s0, sem_s1) = refs
        c = lax.axis_index("c")
        s = lax.axis_index("s")
        z16 = jnp.zeros((16,), jnp.float32)
        for r in range(128):
            mv[0, r, :] = z16

        def zc(k, car):
            pltpu.sync_copy(mv.at[0, pl.ds(0, 128)],
                            acc.at[pl.ds(s * 6400 + k * 128, 128)])
            return car

        lax.fori_loop(0, 50, zc, 0)
        plsc.subcore_barrier()

        sl = (sem_l0, sem_l1)
        sg = (sem_g0, sem_g1)
        ss = (sem_s0, sem_s1)

        def run_pass(nsup, e_base, tbl, col_off, t_off, acc_off):
            def e_of(i):
                return e_base + i * 256

            def start_load(sp, e0):
                pltpu.async_copy(src_h.at[pl.ds(e0, 256)], sidx.at[sp],
                                 sl[sp])
                for j in range(2):
                    pltpu.async_copy(dst_h.at[pl.ds(e0 + j * 128, 128)],
                                     didx.at[sp, j], sl[sp])
                pltpu.async_copy(wf.at[pl.ds(e0, 256), pl.ds(col_off, 16)],
                                 wv.at[sp], sl[sp])

            def wait_load(sp, e0):
                pltpu.make_async_copy(src_h.at[pl.ds(e0, 256)], sidx.at[sp],
                                      sl[sp]).wait()
                for j in range(2):
                    pltpu.make_async_copy(dst_h.at[pl.ds(e0 + j * 128, 128)],
                                          didx.at[sp, j], sl[sp]).wait()
                pltpu.make_async_copy(
                    wf.at[pl.ds(e0, 256), pl.ds(col_off, 16)],
                    wv.at[sp], sl[sp]).wait()

            def adjust(sp):
                if t_off is not None:
                    for j in range(16):
                        sidx[sp, pl.ds(j * 16, 16)] = (
                            sidx[sp, pl.ds(j * 16, 16)] + t_off)
                if acc_off:
                    for j in range(2):
                        for k in range(8):
                            didx[sp, j, pl.ds(k * 16, 16)] = (
                                didx[sp, j, pl.ds(k * 16, 16)] + acc_off)

            def start_gather(sp):
                for j in range(2):
                    pltpu.async_copy(
                        tbl.at[sidx.at[sp, pl.ds(j * 128, 128)]],
                        rows.at[sp, pl.ds(j * 128, 128)], sg[sp])

            def wait_gather(sp):
                for j in range(2):
                    pltpu.make_async_copy(
                        tbl.at[sidx.at[sp, pl.ds(j * 128, 128)]],
                        rows.at[sp, pl.ds(j * 128, 128)], sg[sp]).wait()

            def wait_scatter(sp):
                for j in range(2):
                    pltpu.make_async_copy(mv.at[sp, pl.ds(j * 128, 128)],
                                          acc.at[didx_s.at[sp, j]],
                                          ss[sp]).wait()

            def compute_scatter(i, sp):
                @pl.when(i >= 2)
                def _():
                    wait_scatter(sp)

                def mul8(r8, car2):
                    r0 = r8 * 8
                    for j in range(8):
                        mv[sp, r0 + j, :] = (rows[sp, r0 + j, :]
                                             * wv[sp, r0 + j, :])
                    return car2

                lax.fori_loop(0, 32, mul8, 0)
                for j in range(2):
                    for k in range(8):
                        didx_s[sp, j, pl.ds(k * 16, 16)] = (
                            didx[sp, j, pl.ds(k * 16, 16)])
                for j in range(2):
                    pltpu.async_copy(mv.at[sp, pl.ds(j * 128, 128)],
                                     acc.at[didx_s.at[sp, j]], ss[sp],
                                     add=True)

            start_load(0, e_of(0))
            wait_load(0, e_of(0))
            adjust(0)
            start_gather(0)
            if nsup > 1:
                start_load(1, e_of(1))

            def step(i, car):
                def iter_for(sp, ot):
                    wait_gather(sp)

                    @pl.when(i < nsup - 1)
                    def _():
                        wait_load(ot, e_of(i + 1))
                        adjust(ot)
                        start_gather(ot)

                    compute_scatter(i, sp)

                    @pl.when(i < nsup - 2)
                    def _():
                        start_load(sp, e_of(i + 2))

                @pl.when(i % 2 == 0)
                def _():
                    iter_for(0, 1)

                @pl.when(i % 2 == 1)
                def _():
                    iter_for(1, 0)

                return car

            lax.fori_loop(0, nsup, step, 0)
            if nsup >= 2:
                wait_scatter(nsup % 2)
            wait_scatter((nsup - 1) % 2)

        fcol = c * 32 if has_half else c * 16
        run_pass(_EPT_FULL // 256, s * _EPT_FULL, tf, fcol, c * _NPAD, 0)
        if has_half:
            run_pass(_EPT_HALF // 256, c * (_EPAD // 2) + s * _EPT_HALF,
                     th, 16, None, _NPAD)
        plsc.subcore_barrier()
        pltpu.sync_copy(acc.at[pl.ds(s * 3200, 3200)],
                        aggF.at[c, pl.ds(s * 3200, 3200)])
        if has_half:
            pltpu.sync_copy(acc.at[pl.ds(_NPAD + s * 3200, 3200)],
                            aggH.at[c, pl.ds(s * 3200, 3200)])

    outs = [jax.ShapeDtypeStruct((2, _NPAD, 16), jnp.float32)]
    if has_half:
        outs.append(jax.ShapeDtypeStruct((2, _NPAD, 16), jnp.float32))
    f = pl.kernel(
        body,
        out_type=tuple(outs) if has_half else outs[0],
        mesh=_mesh(),
        compiler_params=pltpu.CompilerParams(use_tc_tiling_on_sc=False),
        scratch_types=[
            pltpu.VMEM((2, 256), jnp.int32),
            pltpu.VMEM((2, 2, 128), jnp.int32),
            pltpu.VMEM((2, 2, 128), jnp.int32),
            pltpu.VMEM((2, 256, 16), jnp.float32),
            pltpu.VMEM((2, 256, 16), jnp.float32),
            pltpu.VMEM((2, 256, 16), jnp.float32),
            pltpu.VMEM_SHARED((2 * _NPAD, 16), jnp.float32),
            pltpu.SemaphoreType.DMA,
            pltpu.SemaphoreType.DMA,
            pltpu.SemaphoreType.DMA,
            pltpu.SemaphoreType.DMA,
            pltpu.SemaphoreType.DMA,
            pltpu.SemaphoreType.DMA,
        ],
    )
    if has_half:
        return f(tfull, wt, srcp, dstp, thalf)
    return f(tfull, wt, srcp, dstp)


_TN = _EPAD // 128
_BT = 32


def _tc_feats(exyz3, be8):
    def body(e_ref, be_ref, f_ref):
        i = pl.program_id(0)
        x = e_ref[0]
        y = e_ref[1]
        z = e_ref[2]
        r2 = x * x + y * y + z * z + 1e-18
        rinv = lax.rsqrt(r2)
        r = r2 * rinv
        ux = x * rinv
        uy = y * rinv
        uz = z * rinv
        eg = (i * (_BT * 128)
              + lax.broadcasted_iota(jnp.int32, (_BT, 128), 0) * 128
              + lax.broadcasted_iota(jnp.int32, (_BT, 128), 1))
        isb = (eg >= _ER).astype(jnp.float32)
        vm = (eg < _E).astype(jnp.float32)
        be = be_ref[...]
        for k in range(8):
            f_ref[:, k, :] = be[0, k] + isb * (be[1, k] - be[0, k])
        cutm = (r < _CUT).astype(jnp.float32) * (1.0 / 1.12)
        step = _CUT / 9.0
        for k in range(8):
            vk = _CUT * (k + 1) / 9.0
            dd = (r - vk) * (1.0 / step)
            f_ref[:, 8 + k, :] = jnp.exp(-dd * dd) * cutm
        s3 = 3.0 ** 0.5
        s5 = 5.0 ** 0.5
        s15 = 15.0 ** 0.5
        shs = [vm, s3 * ux * vm, s3 * uy * vm, s3 * uz * vm,
               s15 * ux * uy * vm, s15 * uy * uz * vm,
               (s5 / 2.0) * (3.0 * uz * uz - 1.0) * vm,
               s15 * ux * uz * vm,
               (s15 / 2.0) * (ux * ux - uy * uy) * vm]
        for m, p in enumerate(shs):
            f_ref[:, 16 + m, :] = p
        zz = jnp.zeros((_BT, 128), jnp.float32)
        for f in range(25, 32):
            f_ref[:, f, :] = zz

    return pl.pallas_call(
        body,
        grid=(_TN // _BT,),
        in_specs=[pl.BlockSpec((3, _BT, 128), lambda i: (0, i, 0)),
                  pl.BlockSpec((8, 128), lambda i: (0, 0))],
        out_specs=pl.BlockSpec((_BT, 32, 128), lambda i: (i, 0, 0)),
        out_shape=jax.ShapeDtypeStruct((_TN, 32, 128), jnp.float32),
    )(exyz3, be8)


_BE = 2048


def _tc_w(featsR, Ws, Vs):
    def body(f_ref, w0w, w1w, w2w, w3w, v0w, v1w, v2w, v3w,
             o0, o1, o2, o3):
        blk = f_ref[...]
        for ww, vv, oo in ((w0w, v0w, o0), (w1w, v1w, o1),
                           (w2w, v2w, o2), (w3w, v3w, o3)):
            rw = jnp.dot(blk, ww[...], preferred_element_type=jnp.float32)
            gg = jnp.dot(blk, vv[...], preferred_element_type=jnp.float32)
            oo[...] = rw * gg

    def wspec(d):
        return pl.BlockSpec((32, d), lambda i: (0, 0))

    def ospec(d):
        return pl.BlockSpec((_BE, d), lambda i: (i, 0))

    return pl.pallas_call(
        body,
        grid=(_EPAD // _BE,),
        in_specs=[pl.BlockSpec((_BE, 32), lambda i: (i, 0)),
                  wspec(32), wspec(48), wspec(48), wspec(48),
                  wspec(32), wspec(48), wspec(48), wspec(48)],
        out_specs=[ospec(32), ospec(48), ospec(48), ospec(48)],
        out_shape=[jax.ShapeDtypeStruct((_EPAD, 32), jnp.float32),
                   jax.ShapeDtypeStruct((_EPAD, 48), jnp.float32),
                   jax.ShapeDtypeStruct((_EPAD, 48), jnp.float32),
                   jax.ShapeDtypeStruct((_EPAD, 48), jnp.float32)],
    )(featsR, Ws[0], Ws[1], Ws[2], Ws[3], Vs[0], Vs[1], Vs[2], Vs[3])


_BN = 512


def _tc_upd0(aggF0, t0r, Wp, Ws0s):
    def body(aF_ref, t0_ref, wp_ref, ws_ref, hf_ref, th_ref):
        aF = aF_ref[...]
        t0 = t0_ref[...]
        a32 = jnp.concatenate([aF[0], aF[1]], axis=1)
        h032 = jnp.concatenate([t0[0], t0[1]], axis=1)
        pre = (jnp.dot(a32, wp_ref[...], preferred_element_type=jnp.float32)
               + jnp.dot(h032, ws_ref[...], preferred_element_type=jnp.float32))
        h = pre * (1.0 / (1.0 + jnp.exp(-pre)))
        hf_ref[0] = h[:, 0:16]
        hf_ref[1] = jnp.concatenate(
            [h[:, 32:44], jnp.zeros((_BN, 4), jnp.float32)], axis=1)
        th_ref[...] = h[:, 16:32]

    big = pl.BlockSpec((2, _BN, 16), lambda i: (0, i, 0))
    sml = pl.BlockSpec((_BN, 16), lambda i: (i, 0))
    return pl.pallas_call(
        body,
        grid=(_NPAD // _BN,),
        in_specs=[big, big,
                  pl.BlockSpec((32, 44), lambda i: (0, 0)),
                  pl.BlockSpec((32, 44), lambda i: (0, 0))],
        out_specs=[big, sml],
        out_shape=[jax.ShapeDtypeStruct((2, _NPAD, 16), jnp.float32),
                   jax.ShapeDtypeStruct((_NPAD, 16), jnp.float32)],
    )(aggF0, t0r, Wp, Ws0s)


def _tc_updl(aF, aH, hf, th, Wo, Wss, skl, final, Whg=None):
    def body(aF_ref, aH_ref, hf_ref, th_ref, wo_ref, ws_ref, sk_ref, *outs):
        aFv = aF_ref[...]
        aHv = aH_ref[...]
        hfv = hf_ref[...]
        thv = th_ref[...]
        h = jnp.concatenate([hfv[0], thv, hfv[1][:, 0:12]], axis=1)
        a = jnp.concatenate([aFv[0], aHv[0] + aHv[1], aFv[1][:, 0:12]], axis=1)
        pre = (jnp.dot(a, wo_ref[...], preferred_element_type=jnp.float32)
               + jnp.dot(h, ws_ref[...], preferred_element_type=jnp.float32))
        new = pre * (1.0 / (1.0 + jnp.exp(-pre)))
        hn = h + sk_ref[...][0:1, :] * new
        if final:
            wh_ref, out_ref = outs[0], outs[1]
            out_ref[...] = jnp.dot(hn, wh_ref[...],
                                   preferred_element_type=jnp.float32)
        else:
            hfo_ref, tho_ref = outs[0], outs[1]
            hfo_ref[0] = hn[:, 0:16]
            hfo_ref[1] = jnp.concatenate(
                [hn[:, 32:44], jnp.zeros((_BN, 4), jnp.float32)], axis=1)
            tho_ref[...] = hn[:, 16:32]

    big = pl.BlockSpec((2, _BN, 16), lambda i: (0, i, 0))
    sml = pl.BlockSpec((_BN, 16), lambda i: (i, 0))
    full44 = pl.BlockSpec((44, 44), lambda i: (0, 0))
    in_specs = [big, big, big, sml, full44, full44,
                pl.BlockSpec((8, 44), lambda i: (0, 0))]
    if final:
        in_specs.append(pl.BlockSpec((44, 8), lambda i: (0, 0)))
        return pl.pallas_call(
            body,
            grid=(_NPAD // _BN,),
            in_specs=in_specs,
            out_specs=[pl.BlockSpec((_BN, 8), lambda i: (i, 0))],
            out_shape=[jax.ShapeDtypeStruct((_NPAD, 8), jnp.float32)],
        )(aF, aH, hf, th, Wo, Wss, skl, Whg)[0]
    return pl.pallas_call(
        body,
        grid=(_NPAD // _BN,),
        in_specs=in_specs,
        out_specs=[big, sml],
        out_shape=[jax.ShapeDtypeStruct((2, _NPAD, 16), jnp.float32),
                   jax.ShapeDtypeStruct((_NPAD, 16), jnp.float32)],
    )(aF, aH, hf, th, Wo, Wss, skl)


def kernel(pos, atom_types, bonded_edge_index, radial_edge_index, c_noise,
           atom_emb, bond_emb, w_noise0, Wr0, wsh0, Wproj0, Wself0, Wr, wsh,
           Wout, Wself, noise_w, skip_w, W_head, gain):
    f32 = jnp.float32
    cn = c_noise[0]
    src = jnp.concatenate([radial_edge_index[0],
                           bonded_edge_index[0]]).astype(jnp.int32)
    dst = jnp.concatenate([radial_edge_index[1],
                           bonded_edge_index[1]]).astype(jnp.int32)
    srcp = jnp.pad(src, (0, _EPAD - _E))
    dstp = jnp.pad(dst, (0, _EPAD - _E))
    atp = jnp.pad(atom_types.astype(jnp.int32), (0, _NPAD - _N))
    pos16 = jnp.pad(pos.astype(f32), ((0, 0), (0, 13)))
    embA = jnp.pad(atom_emb[:, 0:16], ((0, 128 - 119), (0, 0)))
    embB = jnp.pad(atom_emb[:, 16:32], ((0, 128 - 119), (0, 0)))
    emb2 = jnp.concatenate([embA, embB], axis=0)

    s0 = 1.0 + cn * w_noise0
    sl = 1.0 + cn * noise_w
    WrS0 = jnp.pad(Wr0 * s0[None, :], ((0, 16), (0, 0)))
    WrSl = jnp.pad(Wr * sl[:, None, :],
                   ((0, 0), (0, 16), (0, 4)))
    Ws = [WrS0, WrSl[0], WrSl[1], WrSl[2]]
    vsh_all = [wsh0[:, 0], wsh[0][:, 0], wsh[1][:, 0], wsh[2][:, 0]]
    Vs = []
    for li, d in enumerate((32, 48, 48, 48)):
        v32 = jnp.pad(vsh_all[li], (16, 7))
        Vs.append(jnp.broadcast_to(v32[:, None], (32, d)))
    be8 = jnp.zeros((8, 128), f32).at[0:2, 0:8].set(bond_emb)
    Ws0s = s0[:, None] * Wself0
    WselfS = sl[:, :, None] * Wself
    sks = jax.nn.sigmoid(cn * skip_w)
    Whg = jnp.pad(W_head * gain, ((0, 0), (0, 5)))

    exyzT, t0full = _sc_prep(pos16, emb2, srcp, dstp, atp)
    exyz3 = exyzT[:, 0:3].T.reshape(3, _TN, 128)
    feats = _tc_feats(exyz3, be8)
    featsR = feats.transpose(0, 2, 1).reshape(_EPAD, 32)
    w0t, w1t, w2t, w3t = _tc_w(featsR, Ws, Vs)
    aggF0 = _sc_conv(t0full, w0t, srcp, dstp)
    hf, th = _tc_upd0(aggF0, t0full.reshape(2, _NPAD, 16), Wproj0, Ws0s)
    out8 = None
    for l, wlt in enumerate((w1t, w2t, w3t)):
        aF, aH = _sc_conv(hf.reshape(2 * _NPAD, 16), wlt, srcp, dstp, th)
        skl = jnp.zeros((8, 44), f32).at[0].set(sks[l])
        if l < 2:
            hf, th = _tc_updl(aF, aH, hf, th, Wout[l], WselfS[l], skl,
                              final=False)
        else:
            out8 = _tc_updl(aF, aH, hf, th, Wout[l], WselfS[l], skl,
                            final=True, Whg=Whg)
    return out8[:_N, 0:3]


# speedup vs baseline: 7.2079x; 1.1706x over previous
import jax
import jax.numpy as jnp
from jax import lax
from jax.experimental import pallas as pl
from jax.experimental.pallas import tpu as pltpu
from jax.experimental.pallas import tpu_sc as plsc

_N = 50000
_EB = 100000
_ER = 800000
_E = 900000
_NPAD = 51200
_EPAD = 901120
_CUT = 5.0
_NC, _NS = 2, 16

_EPW = _EPAD // (_NC * _NS)
_NPW = _NPAD // (_NC * _NS)
_EPT_FULL = _EPAD // _NS
_EPT_HALF = _EPAD // (2 * _NS)


def _mesh():
    return plsc.VectorSubcoreMesh(
        core_axis_name="c", subcore_axis_name="s",
        num_cores=_NC, num_subcores=_NS)


def _sc_prep(pos16, emb2, srcp, dstp, atp):
    def body(pos_h, emb_h, src_h, dst_h, atp_h, exyzT_h, t0_h,
             sidx, didx, ps, pd, st16, aidx, rowsA, rowsB,
             sem, pl0, pl1, pg0, pg1):
        c = lax.axis_index("c")
        s = lax.axis_index("s")
        w = s * _NC + c
        slp = (pl0, pl1)
        sgp = (pg0, pg1)
        nch = _EPW // 128

        def e_of(i):
            return w * _EPW + i * 128

        def start_load(sp, e0):
            pltpu.async_copy(src_h.at[pl.ds(e0, 128)], sidx.at[sp], slp[sp])
            pltpu.async_copy(dst_h.at[pl.ds(e0, 128)], didx.at[sp], slp[sp])

        def wait_load(sp, e0):
            pltpu.make_async_copy(src_h.at[pl.ds(e0, 128)], sidx.at[sp],
                                  slp[sp]).wait()
            pltpu.make_async_copy(dst_h.at[pl.ds(e0, 128)], didx.at[sp],
                                  slp[sp]).wait()

        def start_gather(sp):
            pltpu.async_copy(pos_h.at[sidx.at[sp]], ps.at[sp], sgp[sp])
            pltpu.async_copy(pos_h.at[didx.at[sp]], pd.at[sp], sgp[sp])

        def wait_gather(sp):
            pltpu.make_async_copy(pos_h.at[sidx.at[sp]], ps.at[sp],
                                  sgp[sp]).wait()
            pltpu.make_async_copy(pos_h.at[didx.at[sp]], pd.at[sp],
                                  sgp[sp]).wait()

        def compute_out(sp, e0):
            def sub8(r8, car2):
                r0 = r8 * 8
                for j in range(8):
                    st16[r0 + j, :] = ps[sp, r0 + j, :] - pd[sp, r0 + j, :]
                return car2

            lax.fori_loop(0, 16, sub8, 0)
            pltpu.sync_copy(st16, exyzT_h.at[pl.ds(e0, 128)])

        start_load(0, e_of(0))
        wait_load(0, e_of(0))
        start_gather(0)
        start_load(1, e_of(1))

        def step(i, car):
            def iter_for(sp, ot):
                wait_gather(sp)

                @pl.when(i < nch - 1)
                def _():
                    wait_load(ot, e_of(i + 1))
                    start_gather(ot)

                compute_out(sp, e_of(i))

                @pl.when(i < nch - 2)
                def _():
                    start_load(sp, e_of(i + 2))

            @pl.when(i % 2 == 0)
            def _():
                iter_for(0, 1)

            @pl.when(i % 2 == 1)
            def _():
                iter_for(1, 0)

            return car

        lax.fori_loop(0, nch, step, 0)

        def nchunk(k, car):
            n0 = w * _NPW + k * 160
            pltpu.sync_copy(atp_h.at[pl.ds(n0, 160)], aidx)
            pltpu.async_copy(emb_h.at[aidx], rowsA, sem).wait()
            for j in range(10):
                aidx[pl.ds(j * 16, 16)] = aidx[pl.ds(j * 16, 16)] + 128
            pltpu.async_copy(emb_h.at[aidx], rowsB, sem).wait()
            pltpu.sync_copy(rowsA, t0_h.at[pl.ds(n0, 160)])
            pltpu.sync_copy(rowsB, t0_h.at[pl.ds(_NPAD + n0, 160)])
            return car

        lax.fori_loop(0, _NPW // 160, nchunk, 0)

    f = pl.kernel(
        body,
        out_type=(jax.ShapeDtypeStruct((_EPAD, 16), jnp.float32),
                  jax.ShapeDtypeStruct((2 * _NPAD, 16), jnp.float32)),
        mesh=_mesh(),
        compiler_params=pltpu.CompilerParams(use_tc_tiling_on_sc=False),
        scratch_types=[
            pltpu.VMEM((2, 128), jnp.int32),
            pltpu.VMEM((2, 128), jnp.int32),
            pltpu.VMEM((2, 128, 16), jnp.float32),
            pltpu.VMEM((2, 128, 16), jnp.float32),
            pltpu.VMEM((128, 16), jnp.float32),
            pltpu.VMEM((160,), jnp.int32),
            pltpu.VMEM((160, 16), jnp.float32),
            pltpu.VMEM((160, 16), jnp.float32),
            pltpu.SemaphoreType.DMA,
            pltpu.SemaphoreType.DMA,
            pltpu.SemaphoreType.DMA,
            pltpu.SemaphoreType.DMA,
            pltpu.SemaphoreType.DMA,
        ],
    )
    return f(pos16, emb2, srcp, dstp, atp)


def _sc_conv(tfull, wt, wbase, srcp, dstp, thalf=None):
    has_half = thalf is not None

    def body(*refs):
        if has_half:
            (tf, wf, src_h, dst_h, th, aggF, aggH,
             sidx, didx, didx_s, rows, wv, mv, acc,
             sem_l0, sem_l1, sem_g0, sem_g1, sem_s0, sem_s1) = refs
        else:
            (tf, wf, src_h, dst_h, aggF,
             sidx, didx, didx_s, rows, wv, mv, acc,
             sem_l0, sem_l1, sem_g0, sem_g1, sem_s0, sem_s1) = refs
        c = lax.axis_index("c")
        s = lax.axis_index("s")
        z16 = jnp.zeros((16,), jnp.float32)
        for r in range(128):
            mv[0, r, :] = z16

        def zc(k, car):
            pltpu.sync_copy(mv.at[0, pl.ds(0, 128)],
                            acc.at[pl.ds(s * 6400 + k * 128, 128)])
            return car

        lax.fori_loop(0, 50, zc, 0)
        plsc.subcore_barrier()

        sl = (sem_l0, sem_l1)
        sg = (sem_g0, sem_g1)
        ss = (sem_s0, sem_s1)

        def run_pass(nsup, e_base, tbl, col_off, t_off, acc_off):
            def e_of(i):
                return e_base + i * 256

            def start_load(sp, e0):
                pltpu.async_copy(src_h.at[pl.ds(e0, 256)], sidx.at[sp],
                                 sl[sp])
                for j in range(2):
                    pltpu.async_copy(dst_h.at[pl.ds(e0 + j * 128, 128)],
                                     didx.at[sp, j], sl[sp])
                pltpu.async_copy(wf.at[pl.ds(e0, 256), pl.ds(col_off, 16)],
                                 wv.at[sp], sl[sp])

            def wait_load(sp, e0):
                pltpu.make_async_copy(src_h.at[pl.ds(e0, 256)], sidx.at[sp],
                                      sl[sp]).wait()
                for j in range(2):
                    pltpu.make_async_copy(dst_h.at[pl.ds(e0 + j * 128, 128)],
                                          didx.at[sp, j], sl[sp]).wait()
                pltpu.make_async_copy(
                    wf.at[pl.ds(e0, 256), pl.ds(col_off, 16)],
                    wv.at[sp], sl[sp]).wait()

            def adjust(sp):
                if t_off is not None:
                    for j in range(16):
                        sidx[sp, pl.ds(j * 16, 16)] = (
                            sidx[sp, pl.ds(j * 16, 16)] + t_off)
                if acc_off:
                    for j in range(2):
                        for k in range(8):
                            didx[sp, j, pl.ds(k * 16, 16)] = (
                                didx[sp, j, pl.ds(k * 16, 16)] + acc_off)

            def start_gather(sp):
                for j in range(2):
                    pltpu.async_copy(
                        tbl.at[sidx.at[sp, pl.ds(j * 128, 128)]],
                        rows.at[sp, pl.ds(j * 128, 128)], sg[sp])

            def wait_gather(sp):
                for j in range(2):
                    pltpu.make_async_copy(
                        tbl.at[sidx.at[sp, pl.ds(j * 128, 128)]],
                        rows.at[sp, pl.ds(j * 128, 128)], sg[sp]).wait()

            def wait_scatter(sp):
                for j in range(2):
                    pltpu.make_async_copy(mv.at[sp, pl.ds(j * 128, 128)],
                                          acc.at[didx_s.at[sp, j]],
                                          ss[sp]).wait()

            def compute_scatter(i, sp):
                @pl.when(i >= 2)
                def _():
                    wait_scatter(sp)

                def mul8(r8, car2):
                    r0 = r8 * 8
                    for j in range(8):
                        mv[sp, r0 + j, :] = (rows[sp, r0 + j, :]
                                             * wv[sp, r0 + j, :])
                    return car2

                lax.fori_loop(0, 32, mul8, 0)
                for j in range(2):
                    for k in range(8):
                        didx_s[sp, j, pl.ds(k * 16, 16)] = (
                            didx[sp, j, pl.ds(k * 16, 16)])
                for j in range(2):
                    pltpu.async_copy(mv.at[sp, pl.ds(j * 128, 128)],
                                     acc.at[didx_s.at[sp, j]], ss[sp],
                                     add=True)

            start_load(0, e_of(0))
            wait_load(0, e_of(0))
            adjust(0)
            start_gather(0)
            if nsup > 1:
                start_load(1, e_of(1))

            def step(i, car):
                def iter_for(sp, ot):
                    wait_gather(sp)

                    @pl.when(i < nsup - 1)
                    def _():
                        wait_load(ot, e_of(i + 1))
                        adjust(ot)
                        start_gather(ot)

                    compute_scatter(i, sp)

                    @pl.when(i < nsup - 2)
                    def _():
                        start_load(sp, e_of(i + 2))

                @pl.when(i % 2 == 0)
                def _():
                    iter_for(0, 1)

                @pl.when(i % 2 == 1)
                def _():
                    iter_for(1, 0)

                return car

            lax.fori_loop(0, nsup, step, 0)
            if nsup >= 2:
                wait_scatter(nsup % 2)
            wait_scatter((nsup - 1) % 2)

        fcol = wbase + (c * 32 if has_half else c * 16)
        run_pass(_EPT_FULL // 256, s * _EPT_FULL, tf, fcol, c * _NPAD, 0)
        if has_half:
            run_pass(_EPT_HALF // 256, c * (_EPAD // 2) + s * _EPT_HALF,
                     th, wbase + 16, None, _NPAD)
        plsc.subcore_barrier()
        pltpu.sync_copy(acc.at[pl.ds(s * 3200, 3200)],
                        aggF.at[c, pl.ds(s * 3200, 3200)])
        if has_half:
            pltpu.sync_copy(acc.at[pl.ds(_NPAD + s * 3200, 3200)],
                            aggH.at[c, pl.ds(s * 3200, 3200)])

    outs = [jax.ShapeDtypeStruct((2, _NPAD, 16), jnp.float32)]
    if has_half:
        outs.append(jax.ShapeDtypeStruct((2, _NPAD, 16), jnp.float32))
    f = pl.kernel(
        body,
        out_type=tuple(outs) if has_half else outs[0],
        mesh=_mesh(),
        compiler_params=pltpu.CompilerParams(use_tc_tiling_on_sc=False),
        scratch_types=[
            pltpu.VMEM((2, 256), jnp.int32),
            pltpu.VMEM((2, 2, 128), jnp.int32),
            pltpu.VMEM((2, 2, 128), jnp.int32),
            pltpu.VMEM((2, 256, 16), jnp.float32),
            pltpu.VMEM((2, 256, 16), jnp.float32),
            pltpu.VMEM((2, 256, 16), jnp.float32),
            pltpu.VMEM_SHARED((2 * _NPAD, 16), jnp.float32),
            pltpu.SemaphoreType.DMA,
            pltpu.SemaphoreType.DMA,
            pltpu.SemaphoreType.DMA,
            pltpu.SemaphoreType.DMA,
            pltpu.SemaphoreType.DMA,
            pltpu.SemaphoreType.DMA,
        ],
    )
    if has_half:
        return f(tfull, wt, srcp, dstp, thalf)
    return f(tfull, wt, srcp, dstp)


_TN = _EPAD // 128
_BT = 32


def _tc_feats(exyz3, be8):
    def body(e_ref, be_ref, f_ref):
        i = pl.program_id(0)
        x = e_ref[0]
        y = e_ref[1]
        z = e_ref[2]
        r2 = x * x + y * y + z * z + 1e-18
        rinv = lax.rsqrt(r2)
        r = r2 * rinv
        ux = x * rinv
        uy = y * rinv
        uz = z * rinv
        eg = (i * (_BT * 128)
              + lax.broadcasted_iota(jnp.int32, (_BT, 128), 0) * 128
              + lax.broadcasted_iota(jnp.int32, (_BT, 128), 1))
        isb = (eg >= _ER).astype(jnp.float32)
        vm = (eg < _E).astype(jnp.float32)
        be = be_ref[...]
        for k in range(8):
            f_ref[:, k, :] = be[0, k] + isb * (be[1, k] - be[0, k])
        cutm = (r < _CUT).astype(jnp.float32) * (1.0 / 1.12)
        step = _CUT / 9.0
        for k in range(8):
            vk = _CUT * (k + 1) / 9.0
            dd = (r - vk) * (1.0 / step)
            f_ref[:, 8 + k, :] = jnp.exp(-dd * dd) * cutm
        s3 = 3.0 ** 0.5
        s5 = 5.0 ** 0.5
        s15 = 15.0 ** 0.5
        shs = [vm, s3 * ux * vm, s3 * uy * vm, s3 * uz * vm,
               s15 * ux * uy * vm, s15 * uy * uz * vm,
               (s5 / 2.0) * (3.0 * uz * uz - 1.0) * vm,
               s15 * ux * uz * vm,
               (s15 / 2.0) * (ux * ux - uy * uy) * vm]
        for m, p in enumerate(shs):
            f_ref[:, 16 + m, :] = p
        zz = jnp.zeros((_BT, 128), jnp.float32)
        for f in range(25, 32):
            f_ref[:, f, :] = zz

    return pl.pallas_call(
        body,
        grid=(_TN // _BT,),
        in_specs=[pl.BlockSpec((3, _BT, 128), lambda i: (0, i, 0)),
                  pl.BlockSpec((8, 128), lambda i: (0, 0))],
        out_specs=pl.BlockSpec((_BT, 32, 128), lambda i: (i, 0, 0)),
        out_shape=jax.ShapeDtypeStruct((_TN, 32, 128), jnp.float32),
    )(exyz3, be8)


_BE = 2048


def _tc_w(featsR, Ws, Vs):
    def body(f_ref, wa_ref, wb_ref, va_ref, vb_ref, oa, ob):
        blk = f_ref[...]
        for ww, vv, oo in ((wa_ref, va_ref, oa), (wb_ref, vb_ref, ob)):
            rw = jnp.dot(blk, ww[...], preferred_element_type=jnp.float32)
            gg = jnp.dot(blk, vv[...], preferred_element_type=jnp.float32)
            oo[...] = rw * gg

    wspec = pl.BlockSpec((32, 128), lambda i: (0, 0))
    ospec = pl.BlockSpec((_BE, 128), lambda i: (i, 0))
    return pl.pallas_call(
        body,
        grid=(_EPAD // _BE,),
        in_specs=[pl.BlockSpec((_BE, 32), lambda i: (i, 0)),
                  wspec, wspec, wspec, wspec],
        out_specs=[ospec, ospec],
        out_shape=[jax.ShapeDtypeStruct((_EPAD, 128), jnp.float32),
                   jax.ShapeDtypeStruct((_EPAD, 128), jnp.float32)],
    )(featsR, Ws[0], Ws[1], Vs[0], Vs[1])


_BN = 512


def _tc_upd0(aggF0, t0r, Wp, Ws0s):
    def body(aF_ref, t0_ref, wp_ref, ws_ref, hf_ref, th_ref):
        aF = aF_ref[...]
        t0 = t0_ref[...]
        a32 = jnp.concatenate([aF[0], aF[1]], axis=1)
        h032 = jnp.concatenate([t0[0], t0[1]], axis=1)
        pre = (jnp.dot(a32, wp_ref[...], preferred_element_type=jnp.float32)
               + jnp.dot(h032, ws_ref[...], preferred_element_type=jnp.float32))
        h = pre * (1.0 / (1.0 + jnp.exp(-pre)))
        hf_ref[0] = h[:, 0:16]
        hf_ref[1] = jnp.concatenate(
            [h[:, 32:44], jnp.zeros((_BN, 4), jnp.float32)], axis=1)
        th_ref[...] = h[:, 16:32]

    big = pl.BlockSpec((2, _BN, 16), lambda i: (0, i, 0))
    sml = pl.BlockSpec((_BN, 16), lambda i: (i, 0))
    return pl.pallas_call(
        body,
        grid=(_NPAD // _BN,),
        in_specs=[big, big,
                  pl.BlockSpec((32, 44), lambda i: (0, 0)),
                  pl.BlockSpec((32, 44), lambda i: (0, 0))],
        out_specs=[big, sml],
        out_shape=[jax.ShapeDtypeStruct((2, _NPAD, 16), jnp.float32),
                   jax.ShapeDtypeStruct((_NPAD, 16), jnp.float32)],
    )(aggF0, t0r, Wp, Ws0s)


def _tc_updl(aF, aH, hf, th, Wo, Wss, skl, final, Whg=None):
    def body(aF_ref, aH_ref, hf_ref, th_ref, wo_ref, ws_ref, sk_ref, *outs):
        aFv = aF_ref[...]
        aHv = aH_ref[...]
        hfv = hf_ref[...]
        thv = th_ref[...]
        h = jnp.concatenate([hfv[0], thv, hfv[1][:, 0:12]], axis=1)
        a = jnp.concatenate([aFv[0], aHv[0] + aHv[1], aFv[1][:, 0:12]], axis=1)
        pre = (jnp.dot(a, wo_ref[...], preferred_element_type=jnp.float32)
               + jnp.dot(h, ws_ref[...], preferred_element_type=jnp.float32))
        new = pre * (1.0 / (1.0 + jnp.exp(-pre)))
        hn = h + sk_ref[...][0:1, :] * new
        if final:
            wh_ref, out_ref = outs[0], outs[1]
            out_ref[...] = jnp.dot(hn, wh_ref[...],
                                   preferred_element_type=jnp.float32)
        else:
            hfo_ref, tho_ref = outs[0], outs[1]
            hfo_ref[0] = hn[:, 0:16]
            hfo_ref[1] = jnp.concatenate(
                [hn[:, 32:44], jnp.zeros((_BN, 4), jnp.float32)], axis=1)
            tho_ref[...] = hn[:, 16:32]

    big = pl.BlockSpec((2, _BN, 16), lambda i: (0, i, 0))
    sml = pl.BlockSpec((_BN, 16), lambda i: (i, 0))
    full44 = pl.BlockSpec((44, 44), lambda i: (0, 0))
    in_specs = [big, big, big, sml, full44, full44,
                pl.BlockSpec((8, 44), lambda i: (0, 0))]
    if final:
        in_specs.append(pl.BlockSpec((44, 8), lambda i: (0, 0)))
        return pl.pallas_call(
            body,
            grid=(_NPAD // _BN,),
            in_specs=in_specs,
            out_specs=[pl.BlockSpec((_BN, 8), lambda i: (i, 0))],
            out_shape=[jax.ShapeDtypeStruct((_NPAD, 8), jnp.float32)],
        )(aF, aH, hf, th, Wo, Wss, skl, Whg)[0]
    return pl.pallas_call(
        body,
        grid=(_NPAD // _BN,),
        in_specs=in_specs,
        out_specs=[big, sml],
        out_shape=[jax.ShapeDtypeStruct((2, _NPAD, 16), jnp.float32),
                   jax.ShapeDtypeStruct((_NPAD, 16), jnp.float32)],
    )(aF, aH, hf, th, Wo, Wss, skl)


def kernel(pos, atom_types, bonded_edge_index, radial_edge_index, c_noise,
           atom_emb, bond_emb, w_noise0, Wr0, wsh0, Wproj0, Wself0, Wr, wsh,
           Wout, Wself, noise_w, skip_w, W_head, gain):
    f32 = jnp.float32
    cn = c_noise[0]
    src = jnp.concatenate([radial_edge_index[0],
                           bonded_edge_index[0]]).astype(jnp.int32)
    dst = jnp.concatenate([radial_edge_index[1],
                           bonded_edge_index[1]]).astype(jnp.int32)
    srcp = jnp.pad(src, (0, _EPAD - _E))
    dstp = jnp.pad(dst, (0, _EPAD - _E))
    atp = jnp.pad(atom_types.astype(jnp.int32), (0, _NPAD - _N))
    pos16 = jnp.pad(pos.astype(f32), ((0, 0), (0, 13)))
    embA = jnp.pad(atom_emb[:, 0:16], ((0, 128 - 119), (0, 0)))
    embB = jnp.pad(atom_emb[:, 16:32], ((0, 128 - 119), (0, 0)))
    emb2 = jnp.concatenate([embA, embB], axis=0)

    s0 = 1.0 + cn * w_noise0
    sl = 1.0 + cn * noise_w
    WrS0 = jnp.pad(Wr0 * s0[None, :], ((0, 16), (0, 0)))
    WrSl = jnp.pad(Wr * sl[:, None, :],
                   ((0, 0), (0, 16), (0, 4)))
    WsA = jnp.concatenate([WrS0, WrSl[0], WrSl[1]], axis=1)
    WsB = jnp.pad(WrSl[2], ((0, 0), (0, 80)))
    vsh_all = [wsh0[:, 0], wsh[0][:, 0], wsh[1][:, 0], wsh[2][:, 0]]
    Vb = []
    for li, d in enumerate((32, 48, 48, 48)):
        v32 = jnp.pad(vsh_all[li], (16, 7))
        Vb.append(jnp.broadcast_to(v32[:, None], (32, d)))
    VsA = jnp.concatenate([Vb[0], Vb[1], Vb[2]], axis=1)
    VsB = jnp.pad(Vb[3], ((0, 0), (0, 80)))
    be8 = jnp.zeros((8, 128), f32).at[0:2, 0:8].set(bond_emb)
    Ws0s = s0[:, None] * Wself0
    WselfS = sl[:, :, None] * Wself
    sks = jax.nn.sigmoid(cn * skip_w)
    Whg = jnp.pad(W_head * gain, ((0, 0), (0, 5)))

    exyzT, t0full = _sc_prep(pos16, emb2, srcp, dstp, atp)
    exyz3 = exyzT[:, 0:3].T.reshape(3, _TN, 128)
    feats = _tc_feats(exyz3, be8)
    featsR = feats.transpose(0, 2, 1).reshape(_EPAD, 32)
    wA, wB = _tc_w(featsR, (WsA, WsB), (VsA, VsB))
    aggF0 = _sc_conv(t0full, wA, 0, srcp, dstp)
    hf, th = _tc_upd0(aggF0, t0full.reshape(2, _NPAD, 16), Wproj0, Ws0s)
    out8 = None
    for l, (wlt, wbase) in enumerate(((wA, 32), (wA, 80), (wB, 0))):
        aF, aH = _sc_conv(hf.reshape(2 * _NPAD, 16), wlt, wbase, srcp,
                          dstp, th)
        skl = jnp.zeros((8, 44), f32).at[0].set(sks[l])
        if l < 2:
            hf, th = _tc_updl(aF, aH, hf, th, Wout[l], WselfS[l], skl,
                              final=False)
        else:
            out8 = _tc_updl(aF, aH, hf, th, Wout[l], WselfS[l], skl,
                            final=True, Whg=Whg)
    return out8[:_N, 0:3]


# speedup vs baseline: 7.4900x; 1.0391x over previous
import jax
import jax.numpy as jnp
from jax import lax
from jax.experimental import pallas as pl
from jax.experimental.pallas import tpu as pltpu
from jax.experimental.pallas import tpu_sc as plsc

_N = 50000
_EB = 100000
_ER = 800000
_E = 900000
_NPAD = 51200
_EPAD = 901120
_CUT = 5.0
_NC, _NS = 2, 16

_EPW = _EPAD // (_NC * _NS)
_NPW = _NPAD // (_NC * _NS)
_EPT_FULL = _EPAD // _NS
_EPT_HALF = _EPAD // (2 * _NS)


def _mesh():
    return plsc.VectorSubcoreMesh(
        core_axis_name="c", subcore_axis_name="s",
        num_cores=_NC, num_subcores=_NS)


def _sc_prep(pos16, emb2, srcp, dstp, atp):
    def body(pos_h, emb_h, src_h, dst_h, atp_h, exyzT_h, t0_h,
             sidx, didx, ps, pd, st16, aidx, rowsA, rowsB,
             sem, pl0, pl1, pg0, pg1):
        c = lax.axis_index("c")
        s = lax.axis_index("s")
        w = s * _NC + c
        slp = (pl0, pl1)
        sgp = (pg0, pg1)
        nch = _EPW // 128

        def e_of(i):
            return w * _EPW + i * 128

        def start_load(sp, e0):
            pltpu.async_copy(src_h.at[pl.ds(e0, 128)], sidx.at[sp], slp[sp])
            pltpu.async_copy(dst_h.at[pl.ds(e0, 128)], didx.at[sp], slp[sp])

        def wait_load(sp, e0):
            pltpu.make_async_copy(src_h.at[pl.ds(e0, 128)], sidx.at[sp],
                                  slp[sp]).wait()
            pltpu.make_async_copy(dst_h.at[pl.ds(e0, 128)], didx.at[sp],
                                  slp[sp]).wait()

        def start_gather(sp):
            pltpu.async_copy(pos_h.at[sidx.at[sp]], ps.at[sp], sgp[sp])
            pltpu.async_copy(pos_h.at[didx.at[sp]], pd.at[sp], sgp[sp])

        def wait_gather(sp):
            pltpu.make_async_copy(pos_h.at[sidx.at[sp]], ps.at[sp],
                                  sgp[sp]).wait()
            pltpu.make_async_copy(pos_h.at[didx.at[sp]], pd.at[sp],
                                  sgp[sp]).wait()

        def compute_out(sp, e0):
            def sub8(r8, car2):
                r0 = r8 * 8
                for j in range(8):
                    st16[r0 + j, :] = ps[sp, r0 + j, :] - pd[sp, r0 + j, :]
                return car2

            lax.fori_loop(0, 16, sub8, 0)
            pltpu.sync_copy(st16, exyzT_h.at[pl.ds(e0, 128)])

        start_load(0, e_of(0))
        wait_load(0, e_of(0))
        start_gather(0)
        start_load(1, e_of(1))

        def step(i, car):
            def iter_for(sp, ot):
                wait_gather(sp)

                @pl.when(i < nch - 1)
                def _():
                    wait_load(ot, e_of(i + 1))
                    start_gather(ot)

                compute_out(sp, e_of(i))

                @pl.when(i < nch - 2)
                def _():
                    start_load(sp, e_of(i + 2))

            @pl.when(i % 2 == 0)
            def _():
                iter_for(0, 1)

            @pl.when(i % 2 == 1)
            def _():
                iter_for(1, 0)

            return car

        lax.fori_loop(0, nch, step, 0)

        def nchunk(k, car):
            n0 = w * _NPW + k * 160
            pltpu.sync_copy(atp_h.at[pl.ds(n0, 160)], aidx)
            pltpu.async_copy(emb_h.at[aidx], rowsA, sem).wait()
            for j in range(10):
                aidx[pl.ds(j * 16, 16)] = aidx[pl.ds(j * 16, 16)] + 128
            pltpu.async_copy(emb_h.at[aidx], rowsB, sem).wait()
            pltpu.sync_copy(rowsA, t0_h.at[pl.ds(n0, 160)])
            pltpu.sync_copy(rowsB, t0_h.at[pl.ds(_NPAD + n0, 160)])
            return car

        lax.fori_loop(0, _NPW // 160, nchunk, 0)

    f = pl.kernel(
        body,
        out_type=(jax.ShapeDtypeStruct((_EPAD, 16), jnp.float32),
                  jax.ShapeDtypeStruct((2 * _NPAD, 16), jnp.float32)),
        mesh=_mesh(),
        compiler_params=pltpu.CompilerParams(use_tc_tiling_on_sc=False),
        scratch_types=[
            pltpu.VMEM((2, 128), jnp.int32),
            pltpu.VMEM((2, 128), jnp.int32),
            pltpu.VMEM((2, 128, 16), jnp.float32),
            pltpu.VMEM((2, 128, 16), jnp.float32),
            pltpu.VMEM((128, 16), jnp.float32),
            pltpu.VMEM((160,), jnp.int32),
            pltpu.VMEM((160, 16), jnp.float32),
            pltpu.VMEM((160, 16), jnp.float32),
            pltpu.SemaphoreType.DMA,
            pltpu.SemaphoreType.DMA,
            pltpu.SemaphoreType.DMA,
            pltpu.SemaphoreType.DMA,
            pltpu.SemaphoreType.DMA,
        ],
    )
    return f(pos16, emb2, srcp, dstp, atp)


def _sc_conv(tfull, wt, wbase, srcp, dstp, thalf=None):
    has_half = thalf is not None

    def body(*refs):
        if has_half:
            (tf, wf, src_h, dst_h, th, aggF, aggH,
             sidx, didx, didx_s, rows, wv, mv, acc,
             sem_l0, sem_l1, sem_g0, sem_g1, sem_s0, sem_s1) = refs
        else:
            (tf, wf, src_h, dst_h, aggF,
             sidx, didx, didx_s, rows, wv, mv, acc,
             sem_l0, sem_l1, sem_g0, sem_g1, sem_s0, sem_s1) = refs
        c = lax.axis_index("c")
        s = lax.axis_index("s")
        z16 = jnp.zeros((16,), jnp.float32)
        for r in range(128):
            mv[0, r, :] = z16

        def zc(k, car):
            pltpu.sync_copy(mv.at[0, pl.ds(0, 128)],
                            acc.at[pl.ds(s * 6400 + k * 128, 128)])
            return car

        lax.fori_loop(0, 50, zc, 0)
        plsc.subcore_barrier()

        sl = (sem_l0, sem_l1)
        sg = (sem_g0, sem_g1)
        ss = (sem_s0, sem_s1)

        def run_pass(nsup, e_base, tbl, col_off, t_off, acc_off):
            def e_of(i):
                return e_base + i * 256

            def start_load(sp, e0):
                pltpu.async_copy(src_h.at[pl.ds(e0, 256)], sidx.at[sp],
                                 sl[sp])
                for j in range(2):
                    pltpu.async_copy(dst_h.at[pl.ds(e0 + j * 128, 128)],
                                     didx.at[sp, j], sl[sp])
                pltpu.async_copy(wf.at[pl.ds(e0, 256), pl.ds(col_off, 16)],
                                 wv.at[sp], sl[sp])

            def wait_load(sp, e0):
                pltpu.make_async_copy(src_h.at[pl.ds(e0, 256)], sidx.at[sp],
                                      sl[sp]).wait()
                for j in range(2):
                    pltpu.make_async_copy(dst_h.at[pl.ds(e0 + j * 128, 128)],
                                          didx.at[sp, j], sl[sp]).wait()
                pltpu.make_async_copy(
                    wf.at[pl.ds(e0, 256), pl.ds(col_off, 16)],
                    wv.at[sp], sl[sp]).wait()

            def adjust(sp):
                if t_off is not None:
                    for j in range(16):
                        sidx[sp, pl.ds(j * 16, 16)] = (
                            sidx[sp, pl.ds(j * 16, 16)] + t_off)
                if acc_off:
                    for j in range(2):
                        for k in range(8):
                            didx[sp, j, pl.ds(k * 16, 16)] = (
                                didx[sp, j, pl.ds(k * 16, 16)] + acc_off)

            def start_gather(sp):
                for j in range(2):
                    pltpu.async_copy(
                        tbl.at[sidx.at[sp, pl.ds(j * 128, 128)]],
                        rows.at[sp, pl.ds(j * 128, 128)], sg[sp])

            def wait_gather(sp):
                for j in range(2):
                    pltpu.make_async_copy(
                        tbl.at[sidx.at[sp, pl.ds(j * 128, 128)]],
                        rows.at[sp, pl.ds(j * 128, 128)], sg[sp]).wait()

            def wait_scatter(sp):
                for j in range(2):
                    pltpu.make_async_copy(mv.at[sp, pl.ds(j * 128, 128)],
                                          acc.at[didx_s.at[sp, j]],
                                          ss[sp]).wait()

            def compute_scatter(i, sp):
                @pl.when(i >= 2)
                def _():
                    wait_scatter(sp)

                def mul8(r8, car2):
                    r0 = r8 * 8
                    for j in range(8):
                        mv[sp, r0 + j, :] = (rows[sp, r0 + j, :]
                                             * wv[sp, r0 + j, :])
                    return car2

                lax.fori_loop(0, 32, mul8, 0)
                for j in range(2):
                    for k in range(8):
                        didx_s[sp, j, pl.ds(k * 16, 16)] = (
                            didx[sp, j, pl.ds(k * 16, 16)])
                for j in range(2):
                    pltpu.async_copy(mv.at[sp, pl.ds(j * 128, 128)],
                                     acc.at[didx_s.at[sp, j]], ss[sp],
                                     add=True)

            start_load(0, e_of(0))
            wait_load(0, e_of(0))
            adjust(0)
            start_gather(0)
            if nsup > 1:
                start_load(1, e_of(1))

            def step(i, car):
                def iter_for(sp, ot):
                    wait_gather(sp)

                    @pl.when(i < nsup - 1)
                    def _():
                        wait_load(ot, e_of(i + 1))
                        adjust(ot)
                        start_gather(ot)

                    compute_scatter(i, sp)

                    @pl.when(i < nsup - 2)
                    def _():
                        start_load(sp, e_of(i + 2))

                @pl.when(i % 2 == 0)
                def _():
                    iter_for(0, 1)

                @pl.when(i % 2 == 1)
                def _():
                    iter_for(1, 0)

                return car

            lax.fori_loop(0, nsup, step, 0)
            if nsup >= 2:
                wait_scatter(nsup % 2)
            wait_scatter((nsup - 1) % 2)

        fcol = wbase + (c * 32 if has_half else c * 16)
        run_pass(_EPT_FULL // 256, s * _EPT_FULL, tf, fcol, c * _NPAD, 0)
        if has_half:
            run_pass(_EPT_HALF // 256, c * (_EPAD // 2) + s * _EPT_HALF,
                     th, wbase + 16, None, _NPAD)
        plsc.subcore_barrier()
        pltpu.sync_copy(acc.at[pl.ds(s * 3200, 3200)],
                        aggF.at[c, pl.ds(s * 3200, 3200)])
        if has_half:
            pltpu.sync_copy(acc.at[pl.ds(_NPAD + s * 3200, 3200)],
                            aggH.at[c, pl.ds(s * 3200, 3200)])

    outs = [jax.ShapeDtypeStruct((2, _NPAD, 16), jnp.float32)]
    if has_half:
        outs.append(jax.ShapeDtypeStruct((2, _NPAD, 16), jnp.float32))
    f = pl.kernel(
        body,
        out_type=tuple(outs) if has_half else outs[0],
        mesh=_mesh(),
        compiler_params=pltpu.CompilerParams(use_tc_tiling_on_sc=False),
        scratch_types=[
            pltpu.VMEM((2, 256), jnp.int32),
            pltpu.VMEM((2, 2, 128), jnp.int32),
            pltpu.VMEM((2, 2, 128), jnp.int32),
            pltpu.VMEM((2, 256, 16), jnp.float32),
            pltpu.VMEM((2, 256, 16), jnp.float32),
            pltpu.VMEM((2, 256, 16), jnp.float32),
            pltpu.VMEM_SHARED((2 * _NPAD, 16), jnp.float32),
            pltpu.SemaphoreType.DMA,
            pltpu.SemaphoreType.DMA,
            pltpu.SemaphoreType.DMA,
            pltpu.SemaphoreType.DMA,
            pltpu.SemaphoreType.DMA,
            pltpu.SemaphoreType.DMA,
        ],
    )
    if has_half:
        return f(tfull, wt, srcp, dstp, thalf)
    return f(tfull, wt, srcp, dstp)


_TN = _EPAD // 128
_BT = 32


def _tc_feats(exyz3, be8):
    def body(e_ref, be_ref, f_ref):
        i = pl.program_id(0)
        x = e_ref[0]
        y = e_ref[1]
        z = e_ref[2]
        r2 = x * x + y * y + z * z + 1e-18
        rinv = lax.rsqrt(r2)
        r = r2 * rinv
        ux = x * rinv
        uy = y * rinv
        uz = z * rinv
        eg = (i * (_BT * 128)
              + lax.broadcasted_iota(jnp.int32, (_BT, 128), 0) * 128
              + lax.broadcasted_iota(jnp.int32, (_BT, 128), 1))
        isb = (eg >= _ER).astype(jnp.float32)
        vm = (eg < _E).astype(jnp.float32)
        be = be_ref[...]
        for k in range(8):
            f_ref[:, k, :] = be[0, k] + isb * (be[1, k] - be[0, k])
        cutm = (r < _CUT).astype(jnp.float32) * (1.0 / 1.12)
        step = _CUT / 9.0
        for k in range(8):
            vk = _CUT * (k + 1) / 9.0
            dd = (r - vk) * (1.0 / step)
            f_ref[:, 8 + k, :] = jnp.exp(-dd * dd) * cutm
        s3 = 3.0 ** 0.5
        s5 = 5.0 ** 0.5
        s15 = 15.0 ** 0.5
        shs = [vm, s3 * ux * vm, s3 * uy * vm, s3 * uz * vm,
               s15 * ux * uy * vm, s15 * uy * uz * vm,
               (s5 / 2.0) * (3.0 * uz * uz - 1.0) * vm,
               s15 * ux * uz * vm,
               (s15 / 2.0) * (ux * ux - uy * uy) * vm]
        for m, p in enumerate(shs):
            f_ref[:, 16 + m, :] = p
        zz = jnp.zeros((_BT, 128), jnp.float32)
        for f in range(25, 32):
            f_ref[:, f, :] = zz

    return pl.pallas_call(
        body,
        grid=(_TN // _BT,),
        in_specs=[pl.BlockSpec((3, _BT, 128), lambda i: (0, i, 0)),
                  pl.BlockSpec((8, 128), lambda i: (0, 0))],
        out_specs=pl.BlockSpec((_BT, 32, 128), lambda i: (i, 0, 0)),
        out_shape=jax.ShapeDtypeStruct((_TN, 32, 128), jnp.float32),
    )(exyz3, be8)


_BE = 2048


def _tc_w(featsR, Ws, Vs):
    def body(f_ref, wa_ref, wb_ref, va_ref, vb_ref, oa, ob):
        dn = (((0,), (0,)), ((), ()))
        for ww, vv, oo in ((wa_ref, va_ref, oa), (wb_ref, vb_ref, ob)):
            wm = ww[...]
            vm = vv[...]
            for tt in range(_BE // 128):
                ft = f_ref[tt]
                rw = lax.dot_general(ft, wm, dn,
                                     preferred_element_type=jnp.float32)
                gg = lax.dot_general(ft, vm, dn,
                                     preferred_element_type=jnp.float32)
                oo[pl.ds(tt * 128, 128), :] = rw * gg

    wspec = pl.BlockSpec((32, 128), lambda i: (0, 0))
    ospec = pl.BlockSpec((_BE, 128), lambda i: (i, 0))
    return pl.pallas_call(
        body,
        grid=(_EPAD // _BE,),
        in_specs=[pl.BlockSpec((_BE // 128, 32, 128), lambda i: (i, 0, 0)),
                  wspec, wspec, wspec, wspec],
        out_specs=[ospec, ospec],
        out_shape=[jax.ShapeDtypeStruct((_EPAD, 128), jnp.float32),
                   jax.ShapeDtypeStruct((_EPAD, 128), jnp.float32)],
    )(featsR, Ws[0], Ws[1], Vs[0], Vs[1])


_BN = 512


def _tc_upd0(aggF0, t0r, Wp, Ws0s):
    def body(aF_ref, t0_ref, wp_ref, ws_ref, hf_ref, th_ref):
        aF = aF_ref[...]
        t0 = t0_ref[...]
        a32 = jnp.concatenate([aF[0], aF[1]], axis=1)
        h032 = jnp.concatenate([t0[0], t0[1]], axis=1)
        pre = (jnp.dot(a32, wp_ref[...], preferred_element_type=jnp.float32)
               + jnp.dot(h032, ws_ref[...], preferred_element_type=jnp.float32))
        h = pre * (1.0 / (1.0 + jnp.exp(-pre)))
        hf_ref[0] = h[:, 0:16]
        hf_ref[1] = jnp.concatenate(
            [h[:, 32:44], jnp.zeros((_BN, 4), jnp.float32)], axis=1)
        th_ref[...] = h[:, 16:32]

    big = pl.BlockSpec((2, _BN, 16), lambda i: (0, i, 0))
    sml = pl.BlockSpec((_BN, 16), lambda i: (i, 0))
    return pl.pallas_call(
        body,
        grid=(_NPAD // _BN,),
        in_specs=[big, big,
                  pl.BlockSpec((32, 44), lambda i: (0, 0)),
                  pl.BlockSpec((32, 44), lambda i: (0, 0))],
        out_specs=[big, sml],
        out_shape=[jax.ShapeDtypeStruct((2, _NPAD, 16), jnp.float32),
                   jax.ShapeDtypeStruct((_NPAD, 16), jnp.float32)],
    )(aggF0, t0r, Wp, Ws0s)


def _tc_updl(aF, aH, hf, th, Wo, Wss, skl, final, Whg=None):
    def body(aF_ref, aH_ref, hf_ref, th_ref, wo_ref, ws_ref, sk_ref, *outs):
        aFv = aF_ref[...]
        aHv = aH_ref[...]
        hfv = hf_ref[...]
        thv = th_ref[...]
        h = jnp.concatenate([hfv[0], thv, hfv[1][:, 0:12]], axis=1)
        a = jnp.concatenate([aFv[0], aHv[0] + aHv[1], aFv[1][:, 0:12]], axis=1)
        pre = (jnp.dot(a, wo_ref[...], preferred_element_type=jnp.float32)
               + jnp.dot(h, ws_ref[...], preferred_element_type=jnp.float32))
        new = pre * (1.0 / (1.0 + jnp.exp(-pre)))
        hn = h + sk_ref[...][0:1, :] * new
        if final:
            wh_ref, out_ref = outs[0], outs[1]
            out_ref[...] = jnp.dot(hn, wh_ref[...],
                                   preferred_element_type=jnp.float32)
        else:
            hfo_ref, tho_ref = outs[0], outs[1]
            hfo_ref[0] = hn[:, 0:16]
            hfo_ref[1] = jnp.concatenate(
                [hn[:, 32:44], jnp.zeros((_BN, 4), jnp.float32)], axis=1)
            tho_ref[...] = hn[:, 16:32]

    big = pl.BlockSpec((2, _BN, 16), lambda i: (0, i, 0))
    sml = pl.BlockSpec((_BN, 16), lambda i: (i, 0))
    full44 = pl.BlockSpec((44, 44), lambda i: (0, 0))
    in_specs = [big, big, big, sml, full44, full44,
                pl.BlockSpec((8, 44), lambda i: (0, 0))]
    if final:
        in_specs.append(pl.BlockSpec((44, 8), lambda i: (0, 0)))
        return pl.pallas_call(
            body,
            grid=(_NPAD // _BN,),
            in_specs=in_specs,
            out_specs=[pl.BlockSpec((_BN, 8), lambda i: (i, 0))],
            out_shape=[jax.ShapeDtypeStruct((_NPAD, 8), jnp.float32)],
        )(aF, aH, hf, th, Wo, Wss, skl, Whg)[0]
    return pl.pallas_call(
        body,
        grid=(_NPAD // _BN,),
        in_specs=in_specs,
        out_specs=[big, sml],
        out_shape=[jax.ShapeDtypeStruct((2, _NPAD, 16), jnp.float32),
                   jax.ShapeDtypeStruct((_NPAD, 16), jnp.float32)],
    )(aF, aH, hf, th, Wo, Wss, skl)


def kernel(pos, atom_types, bonded_edge_index, radial_edge_index, c_noise,
           atom_emb, bond_emb, w_noise0, Wr0, wsh0, Wproj0, Wself0, Wr, wsh,
           Wout, Wself, noise_w, skip_w, W_head, gain):
    f32 = jnp.float32
    cn = c_noise[0]
    src = jnp.concatenate([radial_edge_index[0],
                           bonded_edge_index[0]]).astype(jnp.int32)
    dst = jnp.concatenate([radial_edge_index[1],
                           bonded_edge_index[1]]).astype(jnp.int32)
    srcp = jnp.pad(src, (0, _EPAD - _E))
    dstp = jnp.pad(dst, (0, _EPAD - _E))
    atp = jnp.pad(atom_types.astype(jnp.int32), (0, _NPAD - _N))
    pos16 = jnp.pad(pos.astype(f32), ((0, 0), (0, 13)))
    embA = jnp.pad(atom_emb[:, 0:16], ((0, 128 - 119), (0, 0)))
    embB = jnp.pad(atom_emb[:, 16:32], ((0, 128 - 119), (0, 0)))
    emb2 = jnp.concatenate([embA, embB], axis=0)

    s0 = 1.0 + cn * w_noise0
    sl = 1.0 + cn * noise_w
    WrS0 = jnp.pad(Wr0 * s0[None, :], ((0, 16), (0, 0)))
    WrSl = jnp.pad(Wr * sl[:, None, :],
                   ((0, 0), (0, 16), (0, 4)))
    WsA = jnp.concatenate([WrS0, WrSl[0], WrSl[1]], axis=1)
    WsB = jnp.pad(WrSl[2], ((0, 0), (0, 80)))
    vsh_all = [wsh0[:, 0], wsh[0][:, 0], wsh[1][:, 0], wsh[2][:, 0]]
    Vb = []
    for li, d in enumerate((32, 48, 48, 48)):
        v32 = jnp.pad(vsh_all[li], (16, 7))
        Vb.append(jnp.broadcast_to(v32[:, None], (32, d)))
    VsA = jnp.concatenate([Vb[0], Vb[1], Vb[2]], axis=1)
    VsB = jnp.pad(Vb[3], ((0, 0), (0, 80)))
    be8 = jnp.zeros((8, 128), f32).at[0:2, 0:8].set(bond_emb)
    Ws0s = s0[:, None] * Wself0
    WselfS = sl[:, :, None] * Wself
    sks = jax.nn.sigmoid(cn * skip_w)
    Whg = jnp.pad(W_head * gain, ((0, 0), (0, 5)))

    exyzT, t0full = _sc_prep(pos16, emb2, srcp, dstp, atp)
    exyz3 = exyzT[:, 0:3].T.reshape(3, _TN, 128)
    feats = _tc_feats(exyz3, be8)
    wA, wB = _tc_w(feats, (WsA, WsB), (VsA, VsB))
    aggF0 = _sc_conv(t0full, wA, 0, srcp, dstp)
    hf, th = _tc_upd0(aggF0, t0full.reshape(2, _NPAD, 16), Wproj0, Ws0s)
    out8 = None
    for l, (wlt, wbase) in enumerate(((wA, 32), (wA, 80), (wB, 0))):
        aF, aH = _sc_conv(hf.reshape(2 * _NPAD, 16), wlt, wbase, srcp,
                          dstp, th)
        skl = jnp.zeros((8, 44), f32).at[0].set(sks[l])
        if l < 2:
            hf, th = _tc_updl(aF, aH, hf, th, Wout[l], WselfS[l], skl,
                              final=False)
        else:
            out8 = _tc_updl(aF, aH, hf, th, Wout[l], WselfS[l], skl,
                            final=True, Whg=Whg)
    return out8[:_N, 0:3]
